# baseline (speedup 1.0000x reference)
import jax
import jax.numpy as jnp
from jax.experimental import pallas as pl
from jax.experimental.pallas import tpu as pltpu

GAMMA = 0.1
BLOCK_B = 1024
AP = 1024


def _body(s_ref, unif_ref, mask_ref, w_ref, b_ref, valid_col_ref, out_ref):
    logits = jnp.dot(s_ref[...], w_ref[...], preferred_element_type=jnp.float32)
    logits = logits + b_ref[...]
    mx = jnp.max(logits, axis=1, keepdims=True)
    e = jnp.exp(logits - mx)
    denom = jnp.sum(e, axis=1, keepdims=True)
    probs = GAMMA * unif_ref[...] + ((1.0 - GAMMA) / denom) * e
    valid = jnp.logical_or(mask_ref[...] != 0, valid_col_ref[...] != 0)
    probs = jnp.where(valid, probs, 0.0)
    out_ref[...] = probs * (1.0 / jnp.sum(probs, axis=1, keepdims=True))


@jax.jit
def kernel(s, unif, mask, W, b):
    bsz, d = s.shape
    a = W.shape[1]
    pad = AP - a
    unif_p = jnp.pad(unif, ((0, 0), (0, pad)))
    mask_p = jnp.pad(mask, ((0, 0), (0, pad)))
    w_p = jnp.pad(W, ((0, 0), (0, pad)))
    b_p = jnp.pad(b, (0, pad), constant_values=-1e30).reshape(1, AP)
    valid_col = (jnp.arange(AP, dtype=jnp.int32) == a - 1).astype(jnp.int32)
    valid_col = valid_col.reshape(1, AP)
    grid = (bsz // BLOCK_B,)
    out = pl.pallas_call(
        _body,
        grid=grid,
        in_specs=[
            pl.BlockSpec((BLOCK_B, d), lambda i: (i, 0)),
            pl.BlockSpec((BLOCK_B, AP), lambda i: (i, 0)),
            pl.BlockSpec((BLOCK_B, AP), lambda i: (i, 0)),
            pl.BlockSpec((d, AP), lambda i: (0, 0)),
            pl.BlockSpec((1, AP), lambda i: (0, 0)),
            pl.BlockSpec((1, AP), lambda i: (0, 0)),
        ],
        out_specs=pl.BlockSpec((BLOCK_B, AP), lambda i: (i, 0)),
        out_shape=jax.ShapeDtypeStruct((bsz, AP), jnp.float32),
        compiler_params=pltpu.CompilerParams(
            dimension_semantics=("arbitrary",),
        ),
    )(s, unif_p, mask_p, w_p, b_p, valid_col)
    return out[:, :a]


# speedup vs baseline: 1.3881x; 1.3881x over previous
import jax
import jax.numpy as jnp
from jax.experimental import pallas as pl
from jax.experimental.pallas import tpu as pltpu

GAMMA = 0.1
BLOCK_B = 1024


def _body(s_ref, unif_ref, mask_ref, w_ref, b_ref, out_ref):
    logits = jnp.dot(s_ref[...], w_ref[...], preferred_element_type=jnp.float32)
    logits = logits + b_ref[...]
    mx = jnp.max(logits, axis=1, keepdims=True)
    e = jnp.exp(logits - mx)
    denom = jnp.sum(e, axis=1, keepdims=True)
    u = unif_ref[...].astype(jnp.float32)
    probs = GAMMA * u + ((1.0 - GAMMA) / denom) * e
    a = logits.shape[1]
    col = jax.lax.broadcasted_iota(jnp.int32, logits.shape, 1)
    valid = jnp.logical_or(mask_ref[...] != 0, col == a - 1)
    probs = jnp.where(valid, probs, 0.0)
    scaled = probs * (1.0 / jnp.sum(probs, axis=1, keepdims=True))
    out_ref[...] = scaled.astype(jnp.bfloat16)


@jax.jit
def kernel(s, unif, mask, W, b):
    bsz, d = s.shape
    a = W.shape[1]
    unif_bf = unif.astype(jnp.bfloat16)
    mask_i8 = mask.astype(jnp.int8)
    grid = (bsz // BLOCK_B,)
    out = pl.pallas_call(
        _body,
        grid=grid,
        in_specs=[
            pl.BlockSpec((BLOCK_B, d), lambda i: (i, 0)),
            pl.BlockSpec((BLOCK_B, a), lambda i: (i, 0)),
            pl.BlockSpec((BLOCK_B, a), lambda i: (i, 0)),
            pl.BlockSpec((d, a), lambda i: (0, 0)),
            pl.BlockSpec((1, a), lambda i: (0, 0)),
        ],
        out_specs=pl.BlockSpec((BLOCK_B, a), lambda i: (i, 0)),
        out_shape=jax.ShapeDtypeStruct((bsz, a), jnp.bfloat16),
        compiler_params=pltpu.CompilerParams(
            dimension_semantics=("arbitrary",),
        ),
    )(s, unif_bf, mask_i8, W, b.reshape(1, a))
    return out.astype(jnp.float32)
